# Initial kernel scaffold; baseline (speedup 1.0000x reference)
import jax
import jax.numpy as jnp
from jax import lax
from jax.experimental import pallas as pl
from jax.experimental.pallas import tpu as pltpu
from jax.experimental.pallas import tpu_sc as plsc

_NC, _NS = 2, 16
_NW = _NC * _NS
_CH = 128


def _sc_gather(user_r, item_r, eug, eig, eum, eim):
    nch = user_r.shape[1]
    bpw = nch * _CH
    B = _NW * bpw
    F = eug.shape[1]
    DM = eum.shape[1]
    f32 = jnp.float32
    mesh = plsc.VectorSubcoreMesh(core_axis_name="c", subcore_axis_name="s",
                                  num_cores=_NC, num_subcores=_NS)

    def body(user_h, item_h, eug_h, eig_h, eum_h, eim_h,
             ug_o, ig_o, um_o, im_o,
             uidx, iidx, rug, rig, rbig, sem, sem2):
        wid = lax.axis_index("s") * _NC + lax.axis_index("c")
        base = wid * bpw
        pltpu.sync_copy(user_h.at[wid], uidx)
        pltpu.sync_copy(item_h.at[wid], iidx)
        cps = []
        for j in range(nch):
            sl = pl.ds(j * _CH, _CH)
            cps.append(pltpu.async_copy(eug_h.at[uidx.at[j]], rug.at[sl], sem))
            cps.append(pltpu.async_copy(eig_h.at[iidx.at[j]], rig.at[sl], sem))
            cps.append(pltpu.async_copy(eum_h.at[uidx.at[j]], rbig.at[sl], sem))
        for c in cps:
            c.wait()
        pltpu.sync_copy(rbig, um_o.at[pl.ds(base, bpw)])
        cps2 = []
        for j in range(nch):
            sl = pl.ds(j * _CH, _CH)
            cps2.append(pltpu.async_copy(eim_h.at[iidx.at[j]], rbig.at[sl], sem2))
        pltpu.sync_copy(rug, ug_o.at[pl.ds(base, bpw)])
        pltpu.sync_copy(rig, ig_o.at[pl.ds(base, bpw)])
        for c in cps2:
            c.wait()
        pltpu.sync_copy(rbig, im_o.at[pl.ds(base, bpw)])

    k = pl.kernel(
        body,
        out_type=(jax.ShapeDtypeStruct((B, F), f32),
                  jax.ShapeDtypeStruct((B, F), f32),
                  jax.ShapeDtypeStruct((B, DM), f32),
                  jax.ShapeDtypeStruct((B, DM), f32)),
        mesh=mesh,
        scratch_types=[
            pltpu.VMEM((nch, _CH), jnp.int32),
            pltpu.VMEM((nch, _CH), jnp.int32),
            pltpu.VMEM((bpw, F), f32),
            pltpu.VMEM((bpw, F), f32),
            pltpu.VMEM((bpw, DM), f32),
            pltpu.SemaphoreType.DMA,
            pltpu.SemaphoreType.DMA,
        ],
    )
    return k(user_r, item_r, eug, eig, eum, eim)


def _tc_mlp(ug, ig, um, im, W0a, W0b, b0, W1, b1, W2, b2, wpg, wph, bp):
    B, F = ug.shape
    DM = um.shape[1]
    BT = 2048
    f32 = jnp.float32

    def body(ug_r, ig_r, um_r, im_r, W0a_r, W0b_r, b0_r, W1_r, b1_r,
             W2_r, b2_r, wpg_r, wph_r, bp_r, out_r):
        h = jnp.dot(um_r[...], W0a_r[...], preferred_element_type=f32)
        h = h + jnp.dot(im_r[...], W0b_r[...], preferred_element_type=f32)
        h = jnp.maximum(h + b0_r[...], 0.0)
        h = jnp.maximum(
            jnp.dot(h, W1_r[...], preferred_element_type=f32) + b1_r[...], 0.0)
        h = jnp.maximum(
            jnp.dot(h, W2_r[...], preferred_element_type=f32) + b2_r[...], 0.0)
        g = ug_r[...] * ig_r[...]
        p = (jnp.sum(g * wpg_r[...], axis=1, keepdims=True)
             + jnp.sum(h * wph_r[...], axis=1, keepdims=True) + bp_r[0])
        out_r[...] = p

    full = lambda shape: pl.BlockSpec(shape, lambda i: (0, 0))
    out = pl.pallas_call(
        body,
        grid=(B // BT,),
        in_specs=[
            pl.BlockSpec((BT, F), lambda i: (i, 0)),
            pl.BlockSpec((BT, F), lambda i: (i, 0)),
            pl.BlockSpec((BT, DM), lambda i: (i, 0)),
            pl.BlockSpec((BT, DM), lambda i: (i, 0)),
            full((DM, DM)), full((DM, DM)), full((1, DM)),
            full((DM, DM // 2)), full((1, DM // 2)),
            full((DM // 2, DM // 4)), full((1, DM // 4)),
            full((1, F)), full((1, F)),
            pl.BlockSpec(memory_space=pltpu.SMEM),
        ],
        out_specs=pl.BlockSpec((BT, 1), lambda i: (i, 0)),
        out_shape=jax.ShapeDtypeStruct((B, 1), f32),
    )(ug, ig, um, im, W0a, W0b, b0, W1, b1, W2, b2, wpg, wph, bp)
    return out


def kernel(user, item, emb_user_gmf, emb_item_gmf, emb_user_mlp, emb_item_mlp,
           W0, b0, W1, b1, W2, b2, Wp, bp):
    F = emb_user_gmf.shape[1]
    DM = emb_user_mlp.shape[1]
    user_r = user.astype(jnp.int32).reshape(_NW, -1, _CH)
    item_r = item.astype(jnp.int32).reshape(_NW, -1, _CH)
    ug, ig, um, im = _sc_gather(user_r, item_r, emb_user_gmf, emb_item_gmf,
                                emb_user_mlp, emb_item_mlp)
    pred = _tc_mlp(ug, ig, um, im,
                   W0[:DM], W0[DM:], b0.reshape(1, DM),
                   W1, b1.reshape(1, DM // 2),
                   W2, b2.reshape(1, DM // 4),
                   Wp[:F].reshape(1, F), Wp[F:].reshape(1, F),
                   bp)
    return pred.reshape(-1)


# speedup vs baseline: 1.2547x; 1.2547x over previous
import jax
import jax.numpy as jnp
from jax import lax
from jax.experimental import pallas as pl
from jax.experimental.pallas import tpu as pltpu
from jax.experimental.pallas import tpu_sc as plsc

_NC, _NS = 2, 16
_NW = _NC * _NS
_CH = 128


def _sc_gather(user_r, item_r, eug, eig, eum, eim):
    nch = user_r.shape[1]
    bpw = nch * _CH
    B = _NW * bpw
    F = eug.shape[1]
    DM = eum.shape[1]
    f32 = jnp.float32
    mesh = plsc.VectorSubcoreMesh(core_axis_name="c", subcore_axis_name="s",
                                  num_cores=_NC, num_subcores=_NS)

    def body(user_h, item_h, eug_h, eig_h, eum_h, eim_h,
             ug_o, ig_o, um_o, im_o,
             uidx, iidx, rug, rig, rbig, sem, sem2):
        wid = lax.axis_index("s") * _NC + lax.axis_index("c")
        base = wid * bpw
        pltpu.sync_copy(user_h.at[wid], uidx)
        pltpu.sync_copy(item_h.at[wid], iidx)
        cps = []
        for j in range(nch):
            sl = pl.ds(j * _CH, _CH)
            cps.append(pltpu.async_copy(eug_h.at[uidx.at[j]], rug.at[sl], sem))
            cps.append(pltpu.async_copy(eig_h.at[iidx.at[j]], rig.at[sl], sem))
            cps.append(pltpu.async_copy(eum_h.at[uidx.at[j]], rbig.at[sl], sem))
        for c in cps:
            c.wait()
        pltpu.sync_copy(rbig, um_o.at[pl.ds(base, bpw)])
        cps2 = []
        for j in range(nch):
            sl = pl.ds(j * _CH, _CH)
            cps2.append(pltpu.async_copy(eim_h.at[iidx.at[j]], rbig.at[sl], sem2))
        pltpu.sync_copy(rug, ug_o.at[pl.ds(base, bpw)])
        pltpu.sync_copy(rig, ig_o.at[pl.ds(base, bpw)])
        for c in cps2:
            c.wait()
        pltpu.sync_copy(rbig, im_o.at[pl.ds(base, bpw)])

    k = pl.kernel(
        body,
        out_type=(jax.ShapeDtypeStruct((B, F), f32),
                  jax.ShapeDtypeStruct((B, F), f32),
                  jax.ShapeDtypeStruct((B, DM), f32),
                  jax.ShapeDtypeStruct((B, DM), f32)),
        mesh=mesh,
        compiler_params=pltpu.CompilerParams(use_tc_tiling_on_sc=False),
        scratch_types=[
            pltpu.VMEM((nch, _CH), jnp.int32),
            pltpu.VMEM((nch, _CH), jnp.int32),
            pltpu.VMEM((bpw, F), f32),
            pltpu.VMEM((bpw, F), f32),
            pltpu.VMEM((bpw, DM), f32),
            pltpu.SemaphoreType.DMA,
            pltpu.SemaphoreType.DMA,
        ],
    )
    return k(user_r, item_r, eug, eig, eum, eim)


def _tc_mlp(ug, ig, um, im, W0a, W0b, b0, W1, b1, W2, b2, wpg, wph, bp):
    B, F = ug.shape
    DM = um.shape[1]
    BT = 2048
    f32 = jnp.float32

    def body(ug_r, ig_r, um_r, im_r, W0a_r, W0b_r, b0_r, W1_r, b1_r,
             W2_r, b2_r, wpg_r, wph_r, bp_r, out_r):
        h = jnp.dot(um_r[...], W0a_r[...], preferred_element_type=f32)
        h = h + jnp.dot(im_r[...], W0b_r[...], preferred_element_type=f32)
        h = jnp.maximum(h + b0_r[...], 0.0)
        h = jnp.maximum(
            jnp.dot(h, W1_r[...], preferred_element_type=f32) + b1_r[...], 0.0)
        h = jnp.maximum(
            jnp.dot(h, W2_r[...], preferred_element_type=f32) + b2_r[...], 0.0)
        g = ug_r[...] * ig_r[...]
        p = (jnp.sum(g * wpg_r[...], axis=1, keepdims=True)
             + jnp.sum(h * wph_r[...], axis=1, keepdims=True) + bp_r[0])
        out_r[...] = p

    full = lambda shape: pl.BlockSpec(shape, lambda i: (0, 0))
    out = pl.pallas_call(
        body,
        grid=(B // BT,),
        in_specs=[
            pl.BlockSpec((BT, F), lambda i: (i, 0)),
            pl.BlockSpec((BT, F), lambda i: (i, 0)),
            pl.BlockSpec((BT, DM), lambda i: (i, 0)),
            pl.BlockSpec((BT, DM), lambda i: (i, 0)),
            full((DM, DM)), full((DM, DM)), full((1, DM)),
            full((DM, DM // 2)), full((1, DM // 2)),
            full((DM // 2, DM // 4)), full((1, DM // 4)),
            full((1, F)), full((1, F)),
            pl.BlockSpec(memory_space=pltpu.SMEM),
        ],
        out_specs=pl.BlockSpec((BT, 1), lambda i: (i, 0)),
        out_shape=jax.ShapeDtypeStruct((B, 1), f32),
    )(ug, ig, um, im, W0a, W0b, b0, W1, b1, W2, b2, wpg, wph, bp)
    return out


def kernel(user, item, emb_user_gmf, emb_item_gmf, emb_user_mlp, emb_item_mlp,
           W0, b0, W1, b1, W2, b2, Wp, bp):
    F = emb_user_gmf.shape[1]
    DM = emb_user_mlp.shape[1]
    user_r = user.astype(jnp.int32).reshape(_NW, -1, _CH)
    item_r = item.astype(jnp.int32).reshape(_NW, -1, _CH)
    ug, ig, um, im = _sc_gather(user_r, item_r, emb_user_gmf, emb_item_gmf,
                                emb_user_mlp, emb_item_mlp)
    pred = _tc_mlp(ug, ig, um, im,
                   W0[:DM], W0[DM:], b0.reshape(1, DM),
                   W1, b1.reshape(1, DM // 2),
                   W2, b2.reshape(1, DM // 4),
                   Wp[:F].reshape(1, F), Wp[F:].reshape(1, F),
                   bp)
    return pred.reshape(-1)


# speedup vs baseline: 1.2744x; 1.0157x over previous
import jax
import jax.numpy as jnp
from jax import lax
from jax.experimental import pallas as pl
from jax.experimental.pallas import tpu as pltpu
from jax.experimental.pallas import tpu_sc as plsc

_NC, _NS = 2, 16
_NW = _NC * _NS
_CH = 128


def _sc_gather_mlp(user, item, eum, eim):
    B = user.shape[0]
    bpw = B // _NW
    nch = bpw // _CH
    DM = eum.shape[1]
    f32 = jnp.float32
    mesh = plsc.VectorSubcoreMesh(core_axis_name="c", subcore_axis_name="s",
                                  num_cores=_NC, num_subcores=_NS)

    def body(user_h, item_h, eum_h, eim_h, um_o, im_o, uidx, iidx, rbig,
             sem, sem2):
        wid = lax.axis_index("s") * _NC + lax.axis_index("c")
        base = wid * bpw
        pltpu.sync_copy(user_h.at[pl.ds(base, bpw)], uidx)
        pltpu.sync_copy(item_h.at[pl.ds(base, bpw)], iidx)
        cps = []
        for j in range(nch):
            sl = pl.ds(j * _CH, _CH)
            cps.append(pltpu.async_copy(eum_h.at[uidx.at[sl]], rbig.at[sl], sem))
        for c in cps:
            c.wait()
        pltpu.sync_copy(rbig, um_o.at[pl.ds(base, bpw)])
        cps2 = []
        for j in range(nch):
            sl = pl.ds(j * _CH, _CH)
            cps2.append(pltpu.async_copy(eim_h.at[iidx.at[sl]], rbig.at[sl], sem2))
        for c in cps2:
            c.wait()
        pltpu.sync_copy(rbig, im_o.at[pl.ds(base, bpw)])

    k = pl.kernel(
        body,
        out_type=(jax.ShapeDtypeStruct((B, DM), f32),
                  jax.ShapeDtypeStruct((B, DM), f32)),
        mesh=mesh,
        scratch_types=[
            pltpu.VMEM((bpw,), jnp.int32),
            pltpu.VMEM((bpw,), jnp.int32),
            pltpu.VMEM((bpw, DM), f32),
            pltpu.SemaphoreType.DMA,
            pltpu.SemaphoreType.DMA,
        ],
    )
    return k(user, item, eum, eim)


def _sc_gather_gmf(user_r, item_r, eug, eig):
    nch = user_r.shape[1]
    bpw = nch * _CH
    B = _NW * bpw
    F = eug.shape[1]
    f32 = jnp.float32
    mesh = plsc.VectorSubcoreMesh(core_axis_name="c", subcore_axis_name="s",
                                  num_cores=_NC, num_subcores=_NS)

    def body(user_h, item_h, eug_h, eig_h, ug_o, ig_o, uidx, iidx, rug, rig,
             sem):
        wid = lax.axis_index("s") * _NC + lax.axis_index("c")
        base = wid * bpw
        pltpu.sync_copy(user_h.at[wid], uidx)
        pltpu.sync_copy(item_h.at[wid], iidx)
        cps = []
        for j in range(nch):
            sl = pl.ds(j * _CH, _CH)
            cps.append(pltpu.async_copy(eug_h.at[uidx.at[j]], rug.at[sl], sem))
            cps.append(pltpu.async_copy(eig_h.at[iidx.at[j]], rig.at[sl], sem))
        for c in cps:
            c.wait()
        pltpu.sync_copy(rug, ug_o.at[pl.ds(base, bpw)])
        pltpu.sync_copy(rig, ig_o.at[pl.ds(base, bpw)])

    k = pl.kernel(
        body,
        out_type=(jax.ShapeDtypeStruct((B, F), f32),
                  jax.ShapeDtypeStruct((B, F), f32)),
        mesh=mesh,
        compiler_params=pltpu.CompilerParams(use_tc_tiling_on_sc=False),
        scratch_types=[
            pltpu.VMEM((nch, _CH), jnp.int32),
            pltpu.VMEM((nch, _CH), jnp.int32),
            pltpu.VMEM((bpw, F), f32),
            pltpu.VMEM((bpw, F), f32),
            pltpu.SemaphoreType.DMA,
        ],
    )
    return k(user_r, item_r, eug, eig)


def _tc_mlp(ug, ig, um, im, W0a, W0b, b0, W1, b1, W2, b2, wpg, wph, bp):
    B, F = ug.shape
    DM = um.shape[1]
    BT = 2048
    f32 = jnp.float32

    def body(ug_r, ig_r, um_r, im_r, W0a_r, W0b_r, b0_r, W1_r, b1_r,
             W2_r, b2_r, wpg_r, wph_r, bp_r, out_r):
        h = jnp.dot(um_r[...], W0a_r[...], preferred_element_type=f32)
        h = h + jnp.dot(im_r[...], W0b_r[...], preferred_element_type=f32)
        h = jnp.maximum(h + b0_r[...], 0.0)
        h = jnp.maximum(
            jnp.dot(h, W1_r[...], preferred_element_type=f32) + b1_r[...], 0.0)
        h = jnp.maximum(
            jnp.dot(h, W2_r[...], preferred_element_type=f32) + b2_r[...], 0.0)
        g = ug_r[...] * ig_r[...]
        p = (jnp.sum(g * wpg_r[...], axis=1, keepdims=True)
             + jnp.sum(h * wph_r[...], axis=1, keepdims=True) + bp_r[0])
        out_r[...] = p

    full = lambda shape: pl.BlockSpec(shape, lambda i: (0, 0))
    out = pl.pallas_call(
        body,
        grid=(B // BT,),
        in_specs=[
            pl.BlockSpec((BT, F), lambda i: (i, 0)),
            pl.BlockSpec((BT, F), lambda i: (i, 0)),
            pl.BlockSpec((BT, DM), lambda i: (i, 0)),
            pl.BlockSpec((BT, DM), lambda i: (i, 0)),
            full((DM, DM)), full((DM, DM)), full((1, DM)),
            full((DM, DM // 2)), full((1, DM // 2)),
            full((DM // 2, DM // 4)), full((1, DM // 4)),
            full((1, F)), full((1, F)),
            pl.BlockSpec(memory_space=pltpu.SMEM),
        ],
        out_specs=pl.BlockSpec((BT, 1), lambda i: (i, 0)),
        out_shape=jax.ShapeDtypeStruct((B, 1), f32),
    )(ug, ig, um, im, W0a, W0b, b0, W1, b1, W2, b2, wpg, wph, bp)
    return out


def kernel(user, item, emb_user_gmf, emb_item_gmf, emb_user_mlp, emb_item_mlp,
           W0, b0, W1, b1, W2, b2, Wp, bp):
    F = emb_user_gmf.shape[1]
    DM = emb_user_mlp.shape[1]
    user = user.astype(jnp.int32)
    item = item.astype(jnp.int32)
    um, im = _sc_gather_mlp(user, item, emb_user_mlp, emb_item_mlp)
    ug, ig = _sc_gather_gmf(user.reshape(_NW, -1, _CH),
                            item.reshape(_NW, -1, _CH),
                            emb_user_gmf, emb_item_gmf)
    pred = _tc_mlp(ug, ig, um, im,
                   W0[:DM], W0[DM:], b0.reshape(1, DM),
                   W1, b1.reshape(1, DM // 2),
                   W2, b2.reshape(1, DM // 4),
                   Wp[:F].reshape(1, F), Wp[F:].reshape(1, F),
                   bp)
    return pred.reshape(-1)


# speedup vs baseline: 1.5856x; 1.2442x over previous
import jax
import jax.numpy as jnp
from jax import lax
from jax.experimental import pallas as pl
from jax.experimental.pallas import tpu as pltpu
from jax.experimental.pallas import tpu_sc as plsc

_NC, _NS = 2, 16
_NW = _NC * _NS
_CH = 128
_GC = 256


def _sc_gather(user, item, eug, eig, eum, eim):
    B = user.shape[0]
    bpw = B // _NW
    nch = bpw // _CH
    ngc = bpw // _GC
    F = eug.shape[1]
    DM = eum.shape[1]
    f32 = jnp.float32
    mesh = plsc.VectorSubcoreMesh(core_axis_name="c", subcore_axis_name="s",
                                  num_cores=_NC, num_subcores=_NS)

    def body(user_h, item_h, eug_h, eig_h, eum_h, eim_h,
             ug_o, ig_o, um_o, im_o,
             uidx, iidx, rbig, rg, sem, sem2, semg):
        wid = lax.axis_index("s") * _NC + lax.axis_index("c")
        base = wid * bpw
        pltpu.sync_copy(user_h.at[pl.ds(base, bpw)], uidx)
        pltpu.sync_copy(item_h.at[pl.ds(base, bpw)], iidx)
        cps = []
        for j in range(nch):
            sl = pl.ds(j * _CH, _CH)
            cps.append(pltpu.async_copy(eum_h.at[uidx.at[sl]], rbig.at[sl], sem))
        for tbl_h, idx, out_o in ((eug_h, uidx, ug_o), (eig_h, iidx, ig_o)):
            for c in range(ngc):
                def gmf_group(g, carry, c=c, tbl_h=tbl_h, idx=idx):
                    vec = idx[pl.ds(c * _GC + g * 16, 16)]
                    for l in range(16):
                        pltpu.async_copy(tbl_h.at[vec[l]],
                                         rg.at[g * 16 + l], semg)
                    return carry
                lax.fori_loop(0, _GC // 16, gmf_group, 0)
                pltpu.make_async_copy(tbl_h.at[pl.ds(0, _GC)], rg, semg).wait()
                pltpu.sync_copy(rg, out_o.at[pl.ds(base + c * _GC, _GC)])
        for c in cps:
            c.wait()
        pltpu.sync_copy(rbig, um_o.at[pl.ds(base, bpw)])
        cps2 = []
        for j in range(nch):
            sl = pl.ds(j * _CH, _CH)
            cps2.append(pltpu.async_copy(eim_h.at[iidx.at[sl]], rbig.at[sl], sem2))
        for c in cps2:
            c.wait()
        pltpu.sync_copy(rbig, im_o.at[pl.ds(base, bpw)])

    k = pl.kernel(
        body,
        out_type=(jax.ShapeDtypeStruct((B, F), f32),
                  jax.ShapeDtypeStruct((B, F), f32),
                  jax.ShapeDtypeStruct((B, DM), f32),
                  jax.ShapeDtypeStruct((B, DM), f32)),
        mesh=mesh,
        scratch_types=[
            pltpu.VMEM((bpw,), jnp.int32),
            pltpu.VMEM((bpw,), jnp.int32),
            pltpu.VMEM((bpw, DM), f32),
            pltpu.VMEM((_GC, F), f32),
            pltpu.SemaphoreType.DMA,
            pltpu.SemaphoreType.DMA,
            pltpu.SemaphoreType.DMA,
        ],
    )
    return k(user, item, eug, eig, eum, eim)


def _tc_mlp(ug, ig, um, im, W0a, W0b, b0, W1, b1, W2, b2, wpg, wph, bp):
    B, F = ug.shape
    DM = um.shape[1]
    BT = 2048
    f32 = jnp.float32

    def body(ug_r, ig_r, um_r, im_r, W0a_r, W0b_r, b0_r, W1_r, b1_r,
             W2_r, b2_r, wpg_r, wph_r, bp_r, out_r):
        h = jnp.dot(um_r[...], W0a_r[...], preferred_element_type=f32)
        h = h + jnp.dot(im_r[...], W0b_r[...], preferred_element_type=f32)
        h = jnp.maximum(h + b0_r[...], 0.0)
        h = jnp.maximum(
            jnp.dot(h, W1_r[...], preferred_element_type=f32) + b1_r[...], 0.0)
        h = jnp.maximum(
            jnp.dot(h, W2_r[...], preferred_element_type=f32) + b2_r[...], 0.0)
        g = ug_r[...] * ig_r[...]
        p = (jnp.sum(g * wpg_r[...], axis=1, keepdims=True)
             + jnp.sum(h * wph_r[...], axis=1, keepdims=True) + bp_r[0])
        out_r[...] = p

    full = lambda shape: pl.BlockSpec(shape, lambda i: (0, 0))
    out = pl.pallas_call(
        body,
        grid=(B // BT,),
        in_specs=[
            pl.BlockSpec((BT, F), lambda i: (i, 0)),
            pl.BlockSpec((BT, F), lambda i: (i, 0)),
            pl.BlockSpec((BT, DM), lambda i: (i, 0)),
            pl.BlockSpec((BT, DM), lambda i: (i, 0)),
            full((DM, DM)), full((DM, DM)), full((1, DM)),
            full((DM, DM // 2)), full((1, DM // 2)),
            full((DM // 2, DM // 4)), full((1, DM // 4)),
            full((1, F)), full((1, F)),
            pl.BlockSpec(memory_space=pltpu.SMEM),
        ],
        out_specs=pl.BlockSpec((BT, 1), lambda i: (i, 0)),
        out_shape=jax.ShapeDtypeStruct((B, 1), f32),
    )(ug, ig, um, im, W0a, W0b, b0, W1, b1, W2, b2, wpg, wph, bp)
    return out


def kernel(user, item, emb_user_gmf, emb_item_gmf, emb_user_mlp, emb_item_mlp,
           W0, b0, W1, b1, W2, b2, Wp, bp):
    F = emb_user_gmf.shape[1]
    DM = emb_user_mlp.shape[1]
    ug, ig, um, im = _sc_gather(user.astype(jnp.int32), item.astype(jnp.int32),
                                emb_user_gmf, emb_item_gmf,
                                emb_user_mlp, emb_item_mlp)
    pred = _tc_mlp(ug, ig, um, im,
                   W0[:DM], W0[DM:], b0.reshape(1, DM),
                   W1, b1.reshape(1, DM // 2),
                   W2, b2.reshape(1, DM // 4),
                   Wp[:F].reshape(1, F), Wp[F:].reshape(1, F),
                   bp)
    return pred.reshape(-1)
